# baseline (speedup 1.0000x reference)
import functools

import jax
import jax.numpy as jnp
from jax import lax
from jax.experimental import pallas as pl
from jax.experimental.pallas import tpu as pltpu
from jax.experimental.pallas import tpu_sc as plsc

N_POINTS = 100000
CHUNK = 16
N_CHUNKS = N_POINTS // CHUNK
N_WORKERS = 32
CPW = N_CHUNKS // N_WORKERS
MAX_CPW = CPW + 1
PTS_PER_WORKER = MAX_CPW * CHUNK
N_PAD = N_WORKERS * PTS_PER_WORKER

SCALES = ((56, 64, 3), (28, 128, 67), (14, 256, 195), (7, 512, 451))
OUT_COLS = 963


def _corner(v, g):
    i1 = v.astype(jnp.int32)
    f = v - i1.astype(jnp.float32)
    w_hi = f
    w_lo = jnp.where(f > 0.0, 1.0 - f, 0.0)
    i2 = jnp.minimum(i1 + 1, g - 1)
    return i1, i2, w_lo, w_hi


def _tec_kernel(x_hbm, y_hbm, z_hbm, f0_hbm, f1_hbm, f2_hbm, f3_hbm, out_hbm,
                xb, yb, zb, wbuf, idx0, idx1, idx2, idx3, q0, q1, q2, q3,
                outbuf, sem0, sem1, sem2, sem3):
    wid = lax.axis_index("c") * 16 + lax.axis_index("s")
    nw = CPW + jnp.where(wid < 10, 1, 0)
    base_pt = (wid * CPW + jnp.minimum(wid, 10)) * CHUNK
    iota = lax.iota(jnp.int32, CHUNK)
    zeros = jnp.zeros((CHUNK,), jnp.int32)
    feats = (f0_hbm, f1_hbm, f2_hbm, f3_hbm)
    idxs = (idx0, idx1, idx2, idx3)
    qs = (q0, q1, q2, q3)
    sems = (sem0, sem1, sem2, sem3)

    pltpu.sync_copy(x_hbm.at[pl.ds(base_pt, PTS_PER_WORKER)], xb)
    pltpu.sync_copy(y_hbm.at[pl.ds(base_pt, PTS_PER_WORKER)], yb)
    pltpu.sync_copy(z_hbm.at[pl.ds(base_pt, PTS_PER_WORKER)], zb)

    def chunk_body(k, carry):
        lb = k * CHUNK
        xv = xb[pl.ds(lb, CHUNK)]
        yv = yb[pl.ds(lb, CHUNK)]
        zv = zb[pl.ds(lb, CHUNK)]

        h = 112.0 * ((-yv) / (-zv)) + 111.5
        w = 112.0 * (xv / (-zv)) + 111.5
        h = jnp.minimum(jnp.maximum(h, 0.0), 223.0)
        w = jnp.minimum(jnp.maximum(w, 0.0), 223.0)

        for s, (g, c, _off) in enumerate(SCALES):
            ix1, ix2, wx_lo, wx_hi = _corner(h * (g / 224.0), g)
            jy1, jy2, wy_lo, wy_hi = _corner(w * (g / 224.0), g)
            ib = idxs[s]
            ib[pl.ds(0, CHUNK)] = ix1 * g + jy1
            ib[pl.ds(16, CHUNK)] = ix2 * g + jy1
            ib[pl.ds(32, CHUNK)] = ix1 * g + jy2
            ib[pl.ds(48, CHUNK)] = ix2 * g + jy2
            wbuf[pl.ds(s * 64 + 0, CHUNK)] = wx_lo * wy_lo
            wbuf[pl.ds(s * 64 + 16, CHUNK)] = wx_hi * wy_lo
            wbuf[pl.ds(s * 64 + 32, CHUNK)] = wx_lo * wy_hi
            wbuf[pl.ds(s * 64 + 48, CHUNK)] = wx_hi * wy_hi

        handles = [
            pltpu.async_copy(feats[s].at[idxs[s]], qs[s], sems[s])
            for s in range(4)
        ]

        plsc.store_scatter(outbuf, [iota, zeros], xv)
        plsc.store_scatter(outbuf, [iota, zeros + 1], yv)
        plsc.store_scatter(outbuf, [iota, zeros + 2], zv)

        for s, (g, c, off) in enumerate(SCALES):
            handles[s].wait()
            q = qs[s]

            def point_body(p, carry2, q=q, c=c, off=off, s=s):
                wp = zeros + (s * 64 + p)
                w11v = plsc.load_gather(wbuf, [wp])
                w21v = plsc.load_gather(wbuf, [wp + 16])
                w12v = plsc.load_gather(wbuf, [wp + 32])
                w22v = plsc.load_gather(wbuf, [wp + 48])
                prow = zeros + p
                for c0 in range(0, c, CHUNK):
                    v0 = q[p, pl.ds(c0, CHUNK)]
                    v1 = q[16 + p, pl.ds(c0, CHUNK)]
                    v2 = q[32 + p, pl.ds(c0, CHUNK)]
                    v3 = q[48 + p, pl.ds(c0, CHUNK)]
                    acc = w11v * v0 + w21v * v1 + w12v * v2 + w22v * v3
                    plsc.store_scatter(outbuf, [prow, iota + (off + c0)], acc)
                return carry2

            lax.fori_loop(0, CHUNK, point_body, 0)

        gb = base_pt + lb
        pltpu.sync_copy(outbuf, out_hbm.at[pl.ds(gb, CHUNK)])
        return carry

    lax.fori_loop(0, nw, chunk_body, 0)


@jax.jit
def kernel(coord, img_feat_0, img_feat_1, img_feat_2, img_feat_3):
    pad = N_PAD - N_POINTS
    x = jnp.pad(coord[:, 0], (0, pad))
    y = jnp.pad(coord[:, 1], (0, pad))
    z = jnp.pad(coord[:, 2], (0, pad), constant_values=1.0)
    f0 = img_feat_0.reshape(56 * 56, 64)
    f1 = img_feat_1.reshape(28 * 28, 128)
    f2 = img_feat_2.reshape(14 * 14, 256)
    f3 = img_feat_3.reshape(7 * 7, 512)

    run = functools.partial(
        pl.kernel,
        mesh=plsc.VectorSubcoreMesh(core_axis_name="c", subcore_axis_name="s"),
        compiler_params=pltpu.CompilerParams(needs_layout_passes=False,
                                             use_tc_tiling_on_sc=False),
        out_type=jax.ShapeDtypeStruct((N_POINTS, OUT_COLS), jnp.float32),
        scratch_types=[
            pltpu.VMEM((PTS_PER_WORKER,), jnp.float32),
            pltpu.VMEM((PTS_PER_WORKER,), jnp.float32),
            pltpu.VMEM((PTS_PER_WORKER,), jnp.float32),
            pltpu.VMEM((256,), jnp.float32),
            pltpu.VMEM((64,), jnp.int32),
            pltpu.VMEM((64,), jnp.int32),
            pltpu.VMEM((64,), jnp.int32),
            pltpu.VMEM((64,), jnp.int32),
            pltpu.VMEM((64, 64), jnp.float32),
            pltpu.VMEM((64, 128), jnp.float32),
            pltpu.VMEM((64, 256), jnp.float32),
            pltpu.VMEM((64, 512), jnp.float32),
            pltpu.VMEM((CHUNK, OUT_COLS), jnp.float32),
            pltpu.SemaphoreType.DMA,
            pltpu.SemaphoreType.DMA,
            pltpu.SemaphoreType.DMA,
            pltpu.SemaphoreType.DMA,
        ],
    )(_tec_kernel)
    return run(x, y, z, f0, f1, f2, f3)


# speedup vs baseline: 1.0779x; 1.0779x over previous
import functools

import jax
import jax.numpy as jnp
from jax import lax
from jax.experimental import pallas as pl
from jax.experimental.pallas import tpu as pltpu
from jax.experimental.pallas import tpu_sc as plsc

N_POINTS = 100000
CHUNK = 16
N_CHUNKS = N_POINTS // CHUNK
N_WORKERS = 32
CPW = N_CHUNKS // N_WORKERS
MAX_CPW = CPW + 1
PTS_PER_WORKER = MAX_CPW * CHUNK
N_PAD = N_WORKERS * PTS_PER_WORKER

SCALES = ((56, 64, 3), (28, 128, 67), (14, 256, 195), (7, 512, 451))
OUT_COLS = 963


def _corner(v, g):
    i1 = v.astype(jnp.int32)
    f = v - i1.astype(jnp.float32)
    w_hi = f
    w_lo = jnp.where(f > 0.0, 1.0 - f, 0.0)
    i2 = jnp.minimum(i1 + 1, g - 1)
    return i1, i2, w_lo, w_hi


def _tec_kernel(x_hbm, y_hbm, z_hbm, f0_hbm, f1_hbm, f2_hbm, f3_hbm, out_hbm,
                xb, yb, zb, wbuf, idx0, idx1, idx2, idx3, q0, q1, q2, q3,
                outbuf, sem0, sem1, sem2, sem3):
    wid = lax.axis_index("c") * 16 + lax.axis_index("s")
    nw = CPW + jnp.where(wid < 10, 1, 0)
    base_pt = (wid * CPW + jnp.minimum(wid, 10)) * CHUNK
    iota = lax.iota(jnp.int32, CHUNK)
    zeros = jnp.zeros((CHUNK,), jnp.int32)
    feats = (f0_hbm, f1_hbm, f2_hbm, f3_hbm)
    idxs = (idx0, idx1, idx2, idx3)
    qs = (q0, q1, q2, q3)
    sems = (sem0, sem1, sem2, sem3)

    pltpu.sync_copy(x_hbm.at[pl.ds(base_pt, PTS_PER_WORKER)], xb)
    pltpu.sync_copy(y_hbm.at[pl.ds(base_pt, PTS_PER_WORKER)], yb)
    pltpu.sync_copy(z_hbm.at[pl.ds(base_pt, PTS_PER_WORKER)], zb)

    def chunk_body(k, carry):
        lb = k * CHUNK
        xv = xb[pl.ds(lb, CHUNK)]
        yv = yb[pl.ds(lb, CHUNK)]
        zv = zb[pl.ds(lb, CHUNK)]

        h = 112.0 * ((-yv) / (-zv)) + 111.5
        w = 112.0 * (xv / (-zv)) + 111.5
        h = jnp.minimum(jnp.maximum(h, 0.0), 223.0)
        w = jnp.minimum(jnp.maximum(w, 0.0), 223.0)

        for s, (g, c, _off) in enumerate(SCALES):
            ix1, ix2, wx_lo, wx_hi = _corner(h * (g / 224.0), g)
            jy1, jy2, wy_lo, wy_hi = _corner(w * (g / 224.0), g)
            ib = idxs[s]
            ib[pl.ds(0, CHUNK)] = ix1 * g + jy1
            ib[pl.ds(16, CHUNK)] = ix2 * g + jy1
            ib[pl.ds(32, CHUNK)] = ix1 * g + jy2
            ib[pl.ds(48, CHUNK)] = ix2 * g + jy2
            wbuf[pl.ds(s * 64 + 0, CHUNK)] = wx_lo * wy_lo
            wbuf[pl.ds(s * 64 + 16, CHUNK)] = wx_hi * wy_lo
            wbuf[pl.ds(s * 64 + 32, CHUNK)] = wx_lo * wy_hi
            wbuf[pl.ds(s * 64 + 48, CHUNK)] = wx_hi * wy_hi

        handles = [
            pltpu.async_copy(feats[s].at[idxs[s]], qs[s], sems[s])
            for s in range(4)
        ]

        pbase = (iota >> 3) * 8192 + (iota & 7) * 128
        plsc.store_scatter(outbuf, [pbase], xv)
        plsc.store_scatter(outbuf, [pbase + 1], yv)
        plsc.store_scatter(outbuf, [pbase + 2], zv)

        for s, (g, c, off) in enumerate(SCALES):
            handles[s].wait()
            q = qs[s]

            def point_body(p, carry2, q=q, c=c, off=off, s=s):
                wp = zeros + (s * 64 + p)
                w11v = plsc.load_gather(wbuf, [wp])
                w21v = plsc.load_gather(wbuf, [wp + 16])
                w12v = plsc.load_gather(wbuf, [wp + 32])
                w22v = plsc.load_gather(wbuf, [wp + 48])
                pb = (p // 8) * 8192 + (p % 8) * 128
                for c0 in range(0, c, CHUNK):
                    v0 = q[p, pl.ds(c0, CHUNK)]
                    v1 = q[16 + p, pl.ds(c0, CHUNK)]
                    v2 = q[32 + p, pl.ds(c0, CHUNK)]
                    v3 = q[48 + p, pl.ds(c0, CHUNK)]
                    acc = w11v * v0 + w21v * v1 + w12v * v2 + w22v * v3
                    cv = iota + (off + c0)
                    widx = pb + ((cv >> 7) << 10) + (cv & 127)
                    plsc.store_scatter(outbuf, [widx], acc)
                return carry2

            lax.fori_loop(0, CHUNK, point_body, 0)

        gb = base_pt + lb
        pltpu.sync_copy(outbuf, out_hbm.at[pl.ds(gb * 1024, CHUNK * 1024)])
        return carry

    lax.fori_loop(0, nw, chunk_body, 0)


@jax.jit
def kernel(coord, img_feat_0, img_feat_1, img_feat_2, img_feat_3):
    pad = N_PAD - N_POINTS
    x = jnp.pad(coord[:, 0], (0, pad))
    y = jnp.pad(coord[:, 1], (0, pad))
    z = jnp.pad(coord[:, 2], (0, pad), constant_values=1.0)
    f0 = img_feat_0.reshape(56 * 56, 64)
    f1 = img_feat_1.reshape(28 * 28, 128)
    f2 = img_feat_2.reshape(14 * 14, 256)
    f3 = img_feat_3.reshape(7 * 7, 512)

    run = functools.partial(
        pl.kernel,
        mesh=plsc.VectorSubcoreMesh(core_axis_name="c", subcore_axis_name="s"),
        compiler_params=pltpu.CompilerParams(needs_layout_passes=False,
                                             use_tc_tiling_on_sc=False),
        out_type=jax.ShapeDtypeStruct((N_POINTS * 1024,), jnp.float32),
        scratch_types=[
            pltpu.VMEM((PTS_PER_WORKER,), jnp.float32),
            pltpu.VMEM((PTS_PER_WORKER,), jnp.float32),
            pltpu.VMEM((PTS_PER_WORKER,), jnp.float32),
            pltpu.VMEM((256,), jnp.float32),
            pltpu.VMEM((64,), jnp.int32),
            pltpu.VMEM((64,), jnp.int32),
            pltpu.VMEM((64,), jnp.int32),
            pltpu.VMEM((64,), jnp.int32),
            pltpu.VMEM((64, 64), jnp.float32),
            pltpu.VMEM((64, 128), jnp.float32),
            pltpu.VMEM((64, 256), jnp.float32),
            pltpu.VMEM((64, 512), jnp.float32),
            pltpu.VMEM((CHUNK * 1024,), jnp.float32),
            pltpu.SemaphoreType.DMA,
            pltpu.SemaphoreType.DMA,
            pltpu.SemaphoreType.DMA,
            pltpu.SemaphoreType.DMA,
        ],
    )(_tec_kernel)
    flat = run(x, y, z, f0, f1, f2, f3)
    tiles = flat.reshape(N_POINTS // 8, 8, 8, 128)
    out = tiles.transpose(0, 2, 1, 3).reshape(N_POINTS, 1024)
    return out[:, :OUT_COLS]


# speedup vs baseline: 1.1779x; 1.0928x over previous
import functools

import jax
import jax.numpy as jnp
from jax import lax
from jax.experimental import pallas as pl
from jax.experimental.pallas import tpu as pltpu
from jax.experimental.pallas import tpu_sc as plsc

N_POINTS = 100000
CHUNK = 16
N_CHUNKS = N_POINTS // CHUNK
N_WORKERS = 32
CPW = N_CHUNKS // N_WORKERS
MAX_CPW = CPW + 1
PTS_PER_WORKER = MAX_CPW * CHUNK
N_PAD = N_WORKERS * PTS_PER_WORKER

SCALES = ((56, 64, 3), (28, 128, 67), (14, 256, 195), (7, 512, 451))
OUT_COLS = 963


def _corner(v, g):
    i1 = v.astype(jnp.int32)
    f = v - i1.astype(jnp.float32)
    w_hi = f
    w_lo = jnp.where(f > 0.0, 1.0 - f, 0.0)
    i2 = jnp.minimum(i1 + 1, g - 1)
    return i1, i2, w_lo, w_hi


def _tec_kernel(x_hbm, y_hbm, z_hbm, f0_hbm, f1_hbm, f2_hbm, f3_hbm, out_hbm,
                xb, yb, zb, wbuf, idx0, idx1, idx2, idx3, q0, q1, q2, q3,
                outbuf, sem0, sem1, sem2, sem3):
    wid = lax.axis_index("c") * 16 + lax.axis_index("s")
    nw = CPW + jnp.where(wid < 10, 1, 0)
    base_pt = (wid * CPW + jnp.minimum(wid, 10)) * CHUNK
    iota = lax.iota(jnp.int32, CHUNK)
    zeros = jnp.zeros((CHUNK,), jnp.int32)
    feats = (f0_hbm, f1_hbm, f2_hbm, f3_hbm)
    idxs = (idx0, idx1, idx2, idx3)
    qs = (q0, q1, q2, q3)
    sems = (sem0, sem1, sem2, sem3)

    pltpu.sync_copy(x_hbm.at[pl.ds(base_pt, PTS_PER_WORKER)], xb)
    pltpu.sync_copy(y_hbm.at[pl.ds(base_pt, PTS_PER_WORKER)], yb)
    pltpu.sync_copy(z_hbm.at[pl.ds(base_pt, PTS_PER_WORKER)], zb)

    def chunk_body(k, carry):
        lb = k * CHUNK
        xv = xb[pl.ds(lb, CHUNK)]
        yv = yb[pl.ds(lb, CHUNK)]
        zv = zb[pl.ds(lb, CHUNK)]

        h = 112.0 * ((-yv) / (-zv)) + 111.5
        w = 112.0 * (xv / (-zv)) + 111.5
        h = jnp.minimum(jnp.maximum(h, 0.0), 223.0)
        w = jnp.minimum(jnp.maximum(w, 0.0), 223.0)

        for s, (g, c, _off) in enumerate(SCALES):
            ix1, ix2, wx_lo, wx_hi = _corner(h * (g / 224.0), g)
            jy1, jy2, wy_lo, wy_hi = _corner(w * (g / 224.0), g)
            ib = idxs[s]
            ib[pl.ds(0, CHUNK)] = ix1 * g + jy1
            ib[pl.ds(16, CHUNK)] = ix2 * g + jy1
            ib[pl.ds(32, CHUNK)] = ix1 * g + jy2
            ib[pl.ds(48, CHUNK)] = ix2 * g + jy2
            wbuf[pl.ds(s * 64 + 0, CHUNK)] = wx_lo * wy_lo
            wbuf[pl.ds(s * 64 + 16, CHUNK)] = wx_hi * wy_lo
            wbuf[pl.ds(s * 64 + 32, CHUNK)] = wx_lo * wy_hi
            wbuf[pl.ds(s * 64 + 48, CHUNK)] = wx_hi * wy_hi

        handles = [
            pltpu.async_copy(feats[s].at[idxs[s]], qs[s], sems[s])
            for s in range(4)
        ]

        pbase = (iota >> 3) * 8192 + (iota & 7) * 128
        plsc.store_scatter(outbuf, [pbase], xv)
        plsc.store_scatter(outbuf, [pbase + 1], yv)
        plsc.store_scatter(outbuf, [pbase + 2], zv)

        for s, (g, c, off) in enumerate(SCALES):
            handles[s].wait()
            q = qs[s]

            def point_body(p, carry2, q=q, c=c, off=off, s=s):
                wp = zeros + (s * 64 + p)
                w11v = plsc.load_gather(wbuf, [wp])
                w21v = plsc.load_gather(wbuf, [wp + 16])
                w12v = plsc.load_gather(wbuf, [wp + 32])
                w22v = plsc.load_gather(wbuf, [wp + 48])
                pb = (p // 8) * 8192 + (p % 8) * 128
                for c0 in range(0, c, CHUNK):
                    v0 = q[p, pl.ds(c0, CHUNK)]
                    v1 = q[16 + p, pl.ds(c0, CHUNK)]
                    v2 = q[32 + p, pl.ds(c0, CHUNK)]
                    v3 = q[48 + p, pl.ds(c0, CHUNK)]
                    acc = w11v * v0 + w21v * v1 + w12v * v2 + w22v * v3
                    cv = iota + (off + c0)
                    widx = pb + ((cv >> 7) << 10) + (cv & 127)
                    plsc.store_scatter(outbuf, [widx], acc)
                return carry2

            lax.fori_loop(0, CHUNK, point_body, 0)

        gb = base_pt + lb
        pltpu.sync_copy(outbuf, out_hbm.at[pl.ds(gb * 1024, CHUNK * 1024)])
        return carry

    lax.fori_loop(0, nw, chunk_body, 0)


@jax.jit
def kernel(coord, img_feat_0, img_feat_1, img_feat_2, img_feat_3):
    pad = N_PAD - N_POINTS
    x = jnp.pad(coord[:, 0], (0, pad))
    y = jnp.pad(coord[:, 1], (0, pad))
    z = jnp.pad(coord[:, 2], (0, pad), constant_values=1.0)
    f0 = img_feat_0.reshape(56 * 56, 64)
    f1 = img_feat_1.reshape(28 * 28, 128)
    f2 = img_feat_2.reshape(14 * 14, 256)
    f3 = img_feat_3.reshape(7 * 7, 512)

    run = functools.partial(
        pl.kernel,
        mesh=plsc.VectorSubcoreMesh(core_axis_name="c", subcore_axis_name="s"),
        compiler_params=pltpu.CompilerParams(needs_layout_passes=False,
                                             use_tc_tiling_on_sc=False),
        out_type=jax.ShapeDtypeStruct((N_POINTS * 1024,), jnp.float32),
        scratch_types=[
            pltpu.VMEM((PTS_PER_WORKER,), jnp.float32),
            pltpu.VMEM((PTS_PER_WORKER,), jnp.float32),
            pltpu.VMEM((PTS_PER_WORKER,), jnp.float32),
            pltpu.VMEM((256,), jnp.float32),
            pltpu.VMEM((64,), jnp.int32),
            pltpu.VMEM((64,), jnp.int32),
            pltpu.VMEM((64,), jnp.int32),
            pltpu.VMEM((64,), jnp.int32),
            pltpu.VMEM((64, 64), jnp.float32),
            pltpu.VMEM((64, 128), jnp.float32),
            pltpu.VMEM((64, 256), jnp.float32),
            pltpu.VMEM((64, 512), jnp.float32),
            pltpu.VMEM((CHUNK * 1024,), jnp.float32),
            pltpu.SemaphoreType.DMA,
            pltpu.SemaphoreType.DMA,
            pltpu.SemaphoreType.DMA,
            pltpu.SemaphoreType.DMA,
        ],
    )(_tec_kernel)
    flat = run(x, y, z, f0, f1, f2, f3)
    tiles = flat.reshape(N_POINTS // 8, 8, 8, 128)
    out = tiles.transpose(0, 2, 1, 3).reshape(N_POINTS, 1024)
    one = 1.0 + 0.0 * coord[0, 0]
    return out[:, :OUT_COLS] * one
